# baseline (speedup 1.0000x reference)
import jax
import jax.numpy as jnp
from jax.experimental import pallas as pl
from jax.experimental.pallas import tpu as pltpu

_NCHUNKS = 8


def _copy_body(w_hbm, o_hbm, sems):
    rows = w_hbm.shape[0] // _NCHUNKS
    for i in range(_NCHUNKS):
        pltpu.make_async_copy(
            w_hbm.at[pl.ds(i * rows, rows)],
            o_hbm.at[pl.ds(i * rows, rows)],
            sems.at[i],
        ).start()
    for i in range(_NCHUNKS):
        pltpu.make_async_copy(
            w_hbm.at[pl.ds(i * rows, rows)],
            o_hbm.at[pl.ds(i * rows, rows)],
            sems.at[i],
        ).wait()


def kernel(x, emb_weight):
    seq = x.shape[1]
    dim = emb_weight.shape[1]
    out = pl.pallas_call(
        _copy_body,
        in_specs=[pl.BlockSpec(memory_space=pl.ANY)],
        out_specs=pl.BlockSpec(memory_space=pl.ANY),
        out_shape=jax.ShapeDtypeStruct((seq, dim), emb_weight.dtype),
        scratch_shapes=[pltpu.SemaphoreType.DMA((_NCHUNKS,))],
    )(emb_weight)
    return out[None]


# speedup vs baseline: 47.1294x; 47.1294x over previous
import jax
import jax.numpy as jnp
from jax.experimental import pallas as pl
from jax.experimental.pallas import tpu as pltpu

_CH = 512
_RING = 6
_PF = 3


def _copy_body(seq, w_hbm, o_hbm, buf, si, so):
    nch = seq // _CH

    def in_copy(c, b):
        return pltpu.make_async_copy(
            w_hbm.at[pl.ds(c * _CH, _CH)], buf.at[b], si.at[b])

    def out_copy(c, b):
        return pltpu.make_async_copy(
            buf.at[b], o_hbm.at[pl.ds(c * _CH, _CH)], so.at[b])

    outs_unwaited = []
    for c in range(min(_PF, nch)):
        in_copy(c, c % _RING).start()
    for c in range(nch):
        b = c % _RING
        in_copy(c, b).wait()
        out_copy(c, b).start()
        outs_unwaited.append((c, b))
        nxt = c + _PF
        if nxt < nch:
            prev = nxt - _RING
            if prev >= 0:
                out_copy(prev, nxt % _RING).wait()
                outs_unwaited.remove((prev, nxt % _RING))
            in_copy(nxt, nxt % _RING).start()
    for c, b in outs_unwaited:
        out_copy(c, b).wait()


def kernel(x, emb_weight):
    seq = x.shape[1]
    dim = emb_weight.shape[1]
    out = pl.pallas_call(
        lambda w, o, buf, si, so: _copy_body(seq, w, o, buf, si, so),
        in_specs=[pl.BlockSpec(memory_space=pl.ANY)],
        out_specs=pl.BlockSpec(memory_space=pl.ANY),
        out_shape=jax.ShapeDtypeStruct((seq, dim), emb_weight.dtype),
        scratch_shapes=[
            pltpu.VMEM((_RING, _CH, dim), jnp.float32),
            pltpu.SemaphoreType.DMA((_RING,)),
            pltpu.SemaphoreType.DMA((_RING,)),
        ],
    )(emb_weight)
    return out[None]
